# baseline (speedup 1.0000x reference)
import functools

import jax
import jax.numpy as jnp
from jax import lax
from jax.experimental import pallas as pl
from jax.experimental.pallas import tpu as pltpu
from jax.experimental.pallas import tpu_sc as plsc

N = 10000
E = 160000
D = 256
Q = 4
QF = D // Q
NC, NS = 2, 16
CH = 64
NCK = E // CH
ZB = 40
NZB = N // ZB

_F32 = jnp.float32


def _node_mm_body(h_ref, aw, ab, bw, bb, dw, db, ew, eb,
                  ah_ref, dbt_ref, eht_ref):
    hb = h_ref[...]

    def mm(w, b):
        return lax.dot_general(hb, w[...], (((1,), (1,)), ((), ())),
                               preferred_element_type=_F32) + b[...]

    ah_ref[...] = mm(aw, ab)
    bh = mm(bw, bb)
    dh = mm(dw, db)
    eh = mm(ew, eb)
    for q in range(Q):
        dbt_ref[q, :, 0:QF] = dh[:, q * QF:(q + 1) * QF]
        dbt_ref[q, :, QF:2 * QF] = bh[:, q * QF:(q + 1) * QF]
        eht_ref[q, :, 0:QF] = eh[:, q * QF:(q + 1) * QF]
        eht_ref[q, :, QF:2 * QF] = eh[:, q * QF:(q + 1) * QF]


def _edge_mm_body(e_ref, cw, cb, ce_ref):
    ce_ref[...] = lax.dot_general(e_ref[...], cw[...], (((1,), (1,)), ((), ())),
                                  preferred_element_type=_F32) + cb[...]


def _h_out_body(h_ref, ah_ref, acc_ref, ho_ref):
    acc = acc_ref[...]
    ah = ah_ref[...]
    parts = []
    for q in range(Q):
        sm = acc[q, :, 0:QF]
        ss = acc[q, :, QF:2 * QF]
        parts.append(ah[:, q * QF:(q + 1) * QF] + sm / (ss + 1e-6))
    hnew = jnp.concatenate(parts, axis=1)
    ho_ref[...] = h_ref[...] + jnp.maximum(hnew, 0.0)


def _sc_edge_body(dbt_hbm, eht_hbm, ce_hbm, e_hbm, src_hbm, dst_hbm,
                  eout_hbm, acc_hbm,
                  srcv, dstv, dbv, ehv, cev, ev, payv,
                  acc_sh, sem):
    c = lax.axis_index("c")
    s = lax.axis_index("s")

    myzb = (NZB - s + NS - 1) // NS
    myck = (NCK - s + NS - 1) // NS

    for p in range(2):
        q = 2 * c + p
        qoff = q * N

        def zinit(t, _):
            r = t // (2 * QF // 16)
            j = (t % (2 * QF // 16)) * 16
            payv[r, pl.ds(j, 16)] = jnp.zeros((16,), _F32)
            return 0
        lax.fori_loop(0, CH * (2 * QF // 16), zinit, 0)

        def zero_blk(i, _):
            b0 = (s + i * NS) * ZB
            pltpu.sync_copy(payv.at[pl.ds(0, ZB)], acc_sh.at[pl.ds(b0, ZB)])
            return 0
        lax.fori_loop(0, myzb, zero_blk, 0)
        plsc.subcore_barrier()

        def chunk_body(i, _):
            e0 = (s + i * NS) * CH
            pltpu.sync_copy(src_hbm.at[pl.ds(e0, CH)], srcv)

            def adj_src(k, _):
                srcv[pl.ds(k * 16, 16)] = srcv[pl.ds(k * 16, 16)] + qoff
                return 0
            lax.fori_loop(0, CH // 16, adj_src, 0)
            pltpu.async_copy(dbt_hbm.at[srcv], dbv, sem).wait()

            pltpu.sync_copy(dst_hbm.at[pl.ds(e0, CH)], dstv)

            def adj_dst(k, _):
                srcv[pl.ds(k * 16, 16)] = dstv[pl.ds(k * 16, 16)] + qoff
                return 0
            lax.fori_loop(0, CH // 16, adj_dst, 0)
            pltpu.async_copy(eht_hbm.at[srcv], ehv, sem).wait()

            pltpu.sync_copy(ce_hbm.at[pl.ds(e0, CH), pl.ds(q, 1), :], cev)
            pltpu.sync_copy(e_hbm.at[pl.ds(e0, CH), pl.ds(q, 1), :], ev)

            def compute(t, _):
                r = t // (QF // 16)
                j = (t % (QF // 16)) * 16
                dh = dbv[r, pl.ds(j, 16)]
                bh = dbv[r, pl.ds(QF + j, 16)]
                eh = ehv[r, pl.ds(j, 16)]
                ce = cev[r, 0, pl.ds(j, 16)]
                ein = ev[r, 0, pl.ds(j, 16)]
                enew = dh + eh + ce
                sig = 1.0 / (1.0 + jnp.exp(-enew))
                payv[r, pl.ds(j, 16)] = bh * sig
                payv[r, pl.ds(QF + j, 16)] = sig
                ev[r, 0, pl.ds(j, 16)] = ein + jnp.maximum(enew, 0.0)
                return 0
            lax.fori_loop(0, CH * (QF // 16), compute, 0)

            pltpu.sync_copy(ev, eout_hbm.at[pl.ds(e0, CH), pl.ds(q, 1), :])
            pltpu.sync_copy(payv, acc_sh.at[dstv], add=True)
            return 0
        lax.fori_loop(0, myck, chunk_body, 0)
        plsc.subcore_barrier()

        def dump_blk(i, _):
            b0 = (s + i * NS) * ZB
            pltpu.sync_copy(acc_sh.at[pl.ds(b0, ZB)],
                            acc_hbm.at[pl.ds(qoff + b0, ZB)])
            return 0
        lax.fori_loop(0, myzb, dump_blk, 0)


def _sc_edge_call(dbt, eht, ce3, e3, src, dst):
    mesh = plsc.VectorSubcoreMesh(core_axis_name="c", subcore_axis_name="s")
    f = functools.partial(
        pl.kernel,
        out_type=(jax.ShapeDtypeStruct((E, Q, QF), _F32),
                  jax.ShapeDtypeStruct((Q * N, 2 * QF), _F32)),
        mesh=mesh,
        scratch_types=[
            pltpu.VMEM((CH,), jnp.int32),
            pltpu.VMEM((CH,), jnp.int32),
            pltpu.VMEM((CH, 2 * QF), _F32),
            pltpu.VMEM((CH, 2 * QF), _F32),
            pltpu.VMEM((CH, 1, QF), _F32),
            pltpu.VMEM((CH, 1, QF), _F32),
            pltpu.VMEM((CH, 2 * QF), _F32),
            pltpu.VMEM_SHARED((N, 2 * QF), _F32),
            pltpu.SemaphoreType.DMA,
        ],
    )(_sc_edge_body)
    return f(dbt, eht, ce3, e3, src, dst)


def kernel(h, e, edge_index, A_w, A_b, B_w, B_b, C_w, C_b, D_w, D_b, E_w, E_b):
    RN = 400
    RE = 2000

    biases = [b.reshape(1, D) for b in (A_b, B_b, D_b, E_b)]
    w_spec = pl.BlockSpec((D, D), lambda i: (0, 0))
    b_spec = pl.BlockSpec((1, D), lambda i: (0, 0))

    ah, dbt, eht = pl.pallas_call(
        _node_mm_body,
        grid=(N // RN,),
        in_specs=[pl.BlockSpec((RN, D), lambda i: (i, 0)),
                  w_spec, b_spec, w_spec, b_spec, w_spec, b_spec,
                  w_spec, b_spec],
        out_specs=[pl.BlockSpec((RN, D), lambda i: (i, 0)),
                   pl.BlockSpec((Q, RN, 2 * QF), lambda i: (0, i, 0)),
                   pl.BlockSpec((Q, RN, 2 * QF), lambda i: (0, i, 0))],
        out_shape=[jax.ShapeDtypeStruct((N, D), _F32),
                   jax.ShapeDtypeStruct((Q, N, 2 * QF), _F32),
                   jax.ShapeDtypeStruct((Q, N, 2 * QF), _F32)],
    )(h, A_w, biases[0], B_w, biases[1], D_w, biases[2], E_w, biases[3])

    ce = pl.pallas_call(
        _edge_mm_body,
        grid=(E // RE,),
        in_specs=[pl.BlockSpec((RE, D), lambda i: (i, 0)),
                  w_spec, b_spec],
        out_specs=pl.BlockSpec((RE, D), lambda i: (i, 0)),
        out_shape=jax.ShapeDtypeStruct((E, D), _F32),
    )(e, C_w, C_b.reshape(1, D))

    dbt2 = dbt.reshape(Q * N, 2 * QF)
    eht2 = eht.reshape(Q * N, 2 * QF)
    ce3 = ce.reshape(E, Q, QF)
    e3 = e.reshape(E, Q, QF)
    src = edge_index[0]
    dst = edge_index[1]

    eout3, acc = _sc_edge_call(dbt2, eht2, ce3, e3, src, dst)

    h_out = pl.pallas_call(
        _h_out_body,
        grid=(N // RN,),
        in_specs=[pl.BlockSpec((RN, D), lambda i: (i, 0)),
                  pl.BlockSpec((RN, D), lambda i: (i, 0)),
                  pl.BlockSpec((Q, RN, 2 * QF), lambda i: (0, i, 0))],
        out_specs=pl.BlockSpec((RN, D), lambda i: (i, 0)),
        out_shape=jax.ShapeDtypeStruct((N, D), _F32),
    )(h, ah, acc.reshape(Q, N, 2 * QF))

    return (h_out, eout3.reshape(E, D))
